# speedup vs baseline: 5.6244x; 5.6244x over previous
import functools

import jax
import jax.numpy as jnp
from jax import lax
from jax.experimental import pallas as pl
from jax.experimental.pallas import tpu as pltpu
from jax.experimental.pallas import tpu_sc as plsc

N_DST = 10000
D = 128
E_TOTAL = 320000

NUM_CORES = 2
NUM_SUBCORES = 16
NUM_WORKERS = NUM_CORES * NUM_SUBCORES

CHUNK = 128
CHUNKS_PER_WORKER = 79
E_PAD = NUM_WORKERS * CHUNKS_PER_WORKER * CHUNK

ACC_ROWS = 10240
ROWS_PER_TILE = ACC_ROWS // NUM_SUBCORES


def _segsum_body(src_rep_hbm, srcidx_hbm, dstidx_hbm, zeros_hbm, out_hbm,
                 srcidx_v, dstidx_v, buf, acc, sem):
    c = lax.axis_index("c")
    s = lax.axis_index("s")
    wid = c * NUM_SUBCORES + s

    r0 = s * ROWS_PER_TILE
    pltpu.sync_copy(zeros_hbm.at[pl.ds(r0, ROWS_PER_TILE)],
                    acc.at[pl.ds(r0, ROWS_PER_TILE)])
    pltpu.sync_copy(srcidx_hbm.at[wid], srcidx_v)
    pltpu.sync_copy(dstidx_hbm.at[wid], dstidx_v)
    plsc.subcore_barrier()

    @pl.loop(0, CHUNKS_PER_WORKER)
    def _(i):
        pltpu.async_copy(src_rep_hbm.at[srcidx_v.at[i]], buf, sem).wait()
        pltpu.sync_copy(buf, acc.at[dstidx_v.at[i]], add=True)

    plsc.subcore_barrier()
    pltpu.sync_copy(acc.at[pl.ds(r0, ROWS_PER_TILE)],
                    out_hbm.at[c, pl.ds(r0, ROWS_PER_TILE)])


_segsum = functools.partial(
    pl.kernel,
    out_type=jax.ShapeDtypeStruct((NUM_CORES, ACC_ROWS, D), jnp.float32),
    mesh=plsc.VectorSubcoreMesh(core_axis_name="c", subcore_axis_name="s"),
    scratch_types=[
        pltpu.VMEM((CHUNKS_PER_WORKER, CHUNK), jnp.int32),
        pltpu.VMEM((CHUNKS_PER_WORKER, CHUNK), jnp.int32),
        pltpu.VMEM((CHUNK, D), jnp.float32),
        pltpu.VMEM_SHARED((ACC_ROWS, D), jnp.float32),
        pltpu.SemaphoreType.DMA,
    ],
)(_segsum_body)


def _mlp_body(dst_ref, p_ref, w_ref, b_ref, o_ref):
    x1 = dst_ref[...]
    x2 = p_ref[0] + p_ref[1]
    w = w_ref[...]
    acc = lax.dot_general(x1, w[:, :D], (((1,), (1,)), ((), ())),
                          preferred_element_type=jnp.float32)
    acc = acc + lax.dot_general(x2, w[:, D:], (((1,), (1,)), ((), ())),
                                preferred_element_type=jnp.float32)
    o_ref[...] = jnp.maximum(acc + b_ref[...], 0.0)


def kernel(src_rep, dst_rep, edge_index, W, b):
    src = edge_index[0].astype(jnp.int32)
    dst = edge_index[1].astype(jnp.int32)
    e = src.shape[0]
    pad = E_PAD - e
    src_p = jnp.concatenate([src, jnp.zeros((pad,), jnp.int32)])
    dst_p = jnp.concatenate([dst, jnp.full((pad,), N_DST, jnp.int32)])
    src3 = src_p.reshape(NUM_WORKERS, CHUNKS_PER_WORKER, CHUNK)
    dst3 = dst_p.reshape(NUM_WORKERS, CHUNKS_PER_WORKER, CHUNK)
    zeros = jnp.zeros((ACC_ROWS, D), jnp.float32)

    partials = _segsum(src_rep, src3, dst3, zeros)

    n = dst_rep.shape[0]
    block = 1000
    grid = n // block
    out = pl.pallas_call(
        _mlp_body,
        grid=(grid,),
        in_specs=[
            pl.BlockSpec((block, D), lambda i: (i, 0)),
            pl.BlockSpec((NUM_CORES, block, D), lambda i: (0, i, 0)),
            pl.BlockSpec((D, 2 * D), lambda i: (0, 0)),
            pl.BlockSpec((1, D), lambda i: (0, 0)),
        ],
        out_specs=pl.BlockSpec((block, D), lambda i: (i, 0)),
        out_shape=jax.ShapeDtypeStruct((n, D), jnp.float32),
    )(dst_rep, partials, W, b.reshape(1, D))
    return out
